# initial kernel scaffold (unmeasured)
import jax
import jax.numpy as jnp
from jax import lax
from jax.experimental import pallas as pl
from jax.experimental.pallas import tpu as pltpu


def kernel(
    t,
):
    def body(*refs):
        pass

    out_shape = jax.ShapeDtypeStruct(..., jnp.float32)
    return pl.pallas_call(body, out_shape=out_shape)(...)


# baseline (device time: 25966 ns/iter reference)
import jax
import jax.numpy as jnp
from jax import lax
from jax.experimental import pallas as pl
from jax.experimental.pallas import tpu as pltpu

N_DEV = 4


def kernel(t):
    m, n = t.shape

    def body(x_ref, out_ref, comm_ref, send_sems, recv_sems):
        my_pos = lax.axis_index("i")
        left = (my_pos - 1) % N_DEV
        right = (my_pos + 1) % N_DEV

        barrier_sem = pltpu.get_barrier_semaphore()
        for nbr in [left, right]:
            pl.semaphore_signal(
                barrier_sem, inc=1,
                device_id=(nbr,), device_id_type=pl.DeviceIdType.MESH,
            )
        pl.semaphore_wait(barrier_sem, 2)

        comm_ref[0, :, :] = x_ref[:, :]
        acc = x_ref[:, :]

        for h in range(N_DEV - 1):
            rdma = pltpu.make_async_remote_copy(
                src_ref=comm_ref.at[h],
                dst_ref=comm_ref.at[h + 1],
                send_sem=send_sems.at[h],
                recv_sem=recv_sems.at[h],
                device_id=(right,),
                device_id_type=pl.DeviceIdType.MESH,
            )
            rdma.start()
            rdma.wait()
            acc = acc + comm_ref[h + 1, :, :]

        s = acc
        r = jnp.maximum(s, 0.0)
        out_ref[:, :] = jnp.tanh(s) * s * s + r * r * r

    return pl.pallas_call(
        body,
        out_shape=jax.ShapeDtypeStruct((m, n), jnp.float32),
        in_specs=[pl.BlockSpec(memory_space=pltpu.VMEM)],
        out_specs=pl.BlockSpec(memory_space=pltpu.VMEM),
        scratch_shapes=[
            pltpu.VMEM((N_DEV, m, n), jnp.float32),
            pltpu.SemaphoreType.DMA((N_DEV - 1,)),
            pltpu.SemaphoreType.DMA((N_DEV - 1,)),
        ],
        compiler_params=pltpu.CompilerParams(collective_id=0),
    )(t)


# device time: 13757 ns/iter; 1.8875x vs baseline; 1.8875x over previous
import jax
import jax.numpy as jnp
from jax import lax
from jax.experimental import pallas as pl
from jax.experimental.pallas import tpu as pltpu

N_DEV = 4


def kernel(t):
    m, n = t.shape
    half = m // 2

    def body(x_ref, out_ref, comm_ref, sum_ref, send_sems, recv_sems):
        my_pos = lax.axis_index("i")
        left = (my_pos - 1) % N_DEV
        right = (my_pos + 1) % N_DEV
        xp = (N_DEV - 1) - my_pos
        yp = my_pos ^ 1

        barrier_sem = pltpu.get_barrier_semaphore()
        for nbr in [left, right]:
            pl.semaphore_signal(
                barrier_sem, inc=1,
                device_id=(nbr,), device_id_type=pl.DeviceIdType.MESH,
            )
        pl.semaphore_wait(barrier_sem, 2)

        a1 = pltpu.make_async_remote_copy(
            src_ref=x_ref.at[pl.ds(0, half), :],
            dst_ref=comm_ref.at[0],
            send_sem=send_sems.at[0],
            recv_sem=recv_sems.at[0],
            device_id=(xp,),
            device_id_type=pl.DeviceIdType.MESH,
        )
        b1 = pltpu.make_async_remote_copy(
            src_ref=x_ref.at[pl.ds(half, half), :],
            dst_ref=comm_ref.at[1],
            send_sem=send_sems.at[1],
            recv_sem=recv_sems.at[1],
            device_id=(yp,),
            device_id_type=pl.DeviceIdType.MESH,
        )
        a1.start()
        b1.start()

        a1.wait_recv()
        sum_ref[0, :, :] = x_ref[pl.ds(0, half), :] + comm_ref[0, :, :]
        a2 = pltpu.make_async_remote_copy(
            src_ref=sum_ref.at[0],
            dst_ref=comm_ref.at[2],
            send_sem=send_sems.at[2],
            recv_sem=recv_sems.at[2],
            device_id=(yp,),
            device_id_type=pl.DeviceIdType.MESH,
        )
        a2.start()

        b1.wait_recv()
        sum_ref[1, :, :] = x_ref[pl.ds(half, half), :] + comm_ref[1, :, :]
        b2 = pltpu.make_async_remote_copy(
            src_ref=sum_ref.at[1],
            dst_ref=comm_ref.at[3],
            send_sem=send_sems.at[3],
            recv_sem=recv_sems.at[3],
            device_id=(xp,),
            device_id_type=pl.DeviceIdType.MESH,
        )
        b2.start()

        a2.wait_recv()
        s = sum_ref[0, :, :] + comm_ref[2, :, :]
        r = jnp.maximum(s, 0.0)
        out_ref[pl.ds(0, half), :] = jnp.tanh(s) * s * s + r * r * r

        b2.wait_recv()
        s = sum_ref[1, :, :] + comm_ref[3, :, :]
        r = jnp.maximum(s, 0.0)
        out_ref[pl.ds(half, half), :] = jnp.tanh(s) * s * s + r * r * r

        a1.wait_send()
        b1.wait_send()
        a2.wait_send()
        b2.wait_send()

    return pl.pallas_call(
        body,
        out_shape=jax.ShapeDtypeStruct((m, n), jnp.float32),
        in_specs=[pl.BlockSpec(memory_space=pltpu.VMEM)],
        out_specs=pl.BlockSpec(memory_space=pltpu.VMEM),
        scratch_shapes=[
            pltpu.VMEM((4, half, n), jnp.float32),
            pltpu.VMEM((2, half, n), jnp.float32),
            pltpu.SemaphoreType.DMA((4,)),
            pltpu.SemaphoreType.DMA((4,)),
        ],
        compiler_params=pltpu.CompilerParams(collective_id=0),
    )(t)


# device time: 12500 ns/iter; 2.0773x vs baseline; 1.1006x over previous
import jax
import jax.numpy as jnp
from jax import lax
from jax.experimental import pallas as pl
from jax.experimental.pallas import tpu as pltpu

N_DEV = 4
N_CHUNK = 4


def kernel(t):
    m, n = t.shape
    rows = m // N_CHUNK

    def body(x_ref, out_ref, comm_ref, sum_ref, send_sems, recv_sems):
        my_pos = lax.axis_index("i")
        left = (my_pos - 1) % N_DEV
        right = (my_pos + 1) % N_DEV
        xp = (N_DEV - 1) - my_pos
        yp = my_pos ^ 1

        chunks = [
            (0, xp, yp),
            (1, xp, yp),
            (2, yp, xp),
            (3, yp, xp),
        ]

        barrier_sem = pltpu.get_barrier_semaphore()
        for nbr in [left, right]:
            pl.semaphore_signal(
                barrier_sem, inc=1,
                device_id=(nbr,), device_id_type=pl.DeviceIdType.MESH,
            )
        pl.semaphore_wait(barrier_sem, 2)

        p1 = []
        for c, (off, first, _second) in enumerate(chunks):
            rdma = pltpu.make_async_remote_copy(
                src_ref=x_ref.at[pl.ds(off * rows, rows), :],
                dst_ref=comm_ref.at[c],
                send_sem=send_sems.at[c],
                recv_sem=recv_sems.at[c],
                device_id=(first,),
                device_id_type=pl.DeviceIdType.MESH,
            )
            rdma.start()
            p1.append(rdma)

        p2 = []
        for c in (0, 2, 1, 3):
            off, _first, second = chunks[c]
            p1[c].wait_recv()
            sum_ref[c, :, :] = x_ref[pl.ds(off * rows, rows), :] + comm_ref[c, :, :]
            rdma = pltpu.make_async_remote_copy(
                src_ref=sum_ref.at[c],
                dst_ref=comm_ref.at[N_CHUNK + c],
                send_sem=send_sems.at[N_CHUNK + c],
                recv_sem=recv_sems.at[N_CHUNK + c],
                device_id=(second,),
                device_id_type=pl.DeviceIdType.MESH,
            )
            rdma.start()
            p2.append((c, rdma))

        for c, rdma in p2:
            off = chunks[c][0]
            rdma.wait_recv()
            s = sum_ref[c, :, :] + comm_ref[N_CHUNK + c, :, :]
            r = jnp.maximum(s, 0.0)
            out_ref[pl.ds(off * rows, rows), :] = jnp.tanh(s) * s * s + r * r * r

        for rdma in p1:
            rdma.wait_send()
        for _c, rdma in p2:
            rdma.wait_send()

    return pl.pallas_call(
        body,
        out_shape=jax.ShapeDtypeStruct((m, n), jnp.float32),
        in_specs=[pl.BlockSpec(memory_space=pltpu.VMEM)],
        out_specs=pl.BlockSpec(memory_space=pltpu.VMEM),
        scratch_shapes=[
            pltpu.VMEM((2 * N_CHUNK, rows, n), jnp.float32),
            pltpu.VMEM((N_CHUNK, rows, n), jnp.float32),
            pltpu.SemaphoreType.DMA((2 * N_CHUNK,)),
            pltpu.SemaphoreType.DMA((2 * N_CHUNK,)),
        ],
        compiler_params=pltpu.CompilerParams(collective_id=0),
    )(t)
